# baseline (speedup 1.0000x reference)
import functools

import numpy as np

import jax
import jax.numpy as jnp
from jax import lax
from jax.experimental import pallas as pl
from jax.experimental.pallas import tpu as pltpu
from jax.experimental.pallas import tpu_sc as plsc

_NC = 2
_NS = 16
_NW = _NC * _NS

_BI = 64
_BJ = 512
_PK = 4
_CI = 32


def _sc_gather_g2(table, idx4):
    m = idx4.shape[1]
    d = table.shape[1]
    bpw = m // _NS
    mesh = plsc.VectorSubcoreMesh(core_axis_name="c", subcore_axis_name="s")

    @functools.partial(
        pl.kernel,
        mesh=mesh,
        out_type=[
            jax.ShapeDtypeStruct((m, 2 * d), table.dtype),
            jax.ShapeDtypeStruct((m, 2 * d), table.dtype),
        ],
        scratch_types=[
            pltpu.VMEM((bpw,), jnp.int32),
            pltpu.VMEM((bpw, d), table.dtype),
            pltpu.SemaphoreType.DMA,
        ],
    )
    def gather_kernel(table_hbm, idx4_hbm, g_hbm, gp_hbm, idx_v, rows_v, sem):
        half = lax.axis_index("c")
        rbase = lax.axis_index("s") * bpw

        @pl.when(half == 0)
        def _():
            for src, out_hbm in ((0, g_hbm), (2, gp_hbm)):
                pltpu.sync_copy(idx4_hbm.at[src, pl.ds(rbase, bpw)], idx_v)
                pltpu.async_copy(table_hbm.at[idx_v], rows_v, sem).wait()
                pltpu.sync_copy(rows_v,
                                out_hbm.at[pl.ds(rbase, bpw), pl.ds(0, d)])

        @pl.when(half == 1)
        def _():
            for src, out_hbm in ((1, g_hbm), (3, gp_hbm)):
                pltpu.sync_copy(idx4_hbm.at[src, pl.ds(rbase, bpw)], idx_v)
                pltpu.async_copy(table_hbm.at[idx_v], rows_v, sem).wait()
                pltpu.sync_copy(rows_v,
                                out_hbm.at[pl.ds(rbase, bpw), pl.ds(d, d)])

    return gather_kernel(table, idx4)


def _pair_ffnn_body(gi_ref, gjp_ref, wlw_ref, wrbd_ref, b1w_ref, woutp_ref,
                    boutp_ref, out_ref):
    hw = wlw_ref.shape[1]
    ow = woutp_ref.shape[1]
    bjp = gjp_ref.shape[0]
    l = jnp.dot(gi_ref[...], wlw_ref[...], preferred_element_type=jnp.float32)
    l = l + b1w_ref[...]
    r = jnp.dot(gjp_ref[...], wrbd_ref[...], preferred_element_type=jnp.float32)
    for c in range(_BI // _CI):
        lc = l[c * _CI:(c + 1) * _CI]
        h = jnp.maximum(lc[:, None, :] + r[None, :, :], 0.0)
        o = jnp.dot(h.reshape(_CI * bjp, hw), woutp_ref[...],
                    preferred_element_type=jnp.float32)
        o = o + boutp_ref[...]
        out_ref[pl.ds(c * _CI, _CI)] = jnp.transpose(
            o.reshape(_CI, bjp, ow), (0, 2, 1))


def kernel(subword_embeddings, begin_indexes, end_indexes, W1, b1, Wout, bout):
    m = begin_indexes.shape[0]
    d = subword_embeddings.shape[1]
    h_dim = W1.shape[1]
    out_dim = Wout.shape[1]
    pk = _PK
    grp = m // pk

    idx_n = jnp.stack([begin_indexes, end_indexes]).astype(jnp.int32)
    idx_p = idx_n.reshape(2, pk, grp).transpose(0, 2, 1).reshape(2, m)
    idx4 = jnp.concatenate([idx_n, idx_p])
    g, gp = _sc_gather_g2(subword_embeddings, idx4)
    gjp = gp.reshape(grp, pk * 2 * d)

    wl = W1[: 2 * d]
    wr = W1[2 * d:]
    wl_wide = jnp.concatenate([wl] * pk, axis=1)
    wr_bd = jax.scipy.linalg.block_diag(*([wr] * pk))
    wout_bd = jax.scipy.linalg.block_diag(*([Wout] * pk))
    b1_wide = jnp.concatenate([b1] * pk).reshape(1, pk * h_dim)
    bout_wide = jnp.concatenate([bout] * pk).reshape(1, pk * out_dim)
    cols = np.arange(pk * out_dim)
    kk = 8 * (cols // (8 * pk)) + cols % 8
    tt = (cols % (8 * pk)) // 8
    perm = tt * out_dim + kk
    wout_p = wout_bd[:, perm]
    bout_p = bout_wide[:, perm]

    grid = (m // _BI, m // _BJ)
    out = pl.pallas_call(
        _pair_ffnn_body,
        grid=grid,
        in_specs=[
            pl.BlockSpec((_BI, 2 * d), lambda i, j: (i, 0)),
            pl.BlockSpec((_BJ // pk, pk * 2 * d), lambda i, j: (j, 0)),
            pl.BlockSpec((2 * d, pk * h_dim), lambda i, j: (0, 0)),
            pl.BlockSpec((pk * 2 * d, pk * h_dim), lambda i, j: (0, 0)),
            pl.BlockSpec((1, pk * h_dim), lambda i, j: (0, 0)),
            pl.BlockSpec((pk * h_dim, pk * out_dim), lambda i, j: (0, 0)),
            pl.BlockSpec((1, pk * out_dim), lambda i, j: (0, 0)),
        ],
        out_specs=pl.BlockSpec((_BI, pk * out_dim, _BJ // pk),
                               lambda i, j: (i, 0, j)),
        out_shape=jax.ShapeDtypeStruct((m, pk * out_dim, grp), jnp.float32),
    )(g, gjp, wl_wide, wr_bd, b1_wide, wout_p, bout_p)
    out5 = out.reshape(m, out_dim // 8, pk, 8, grp)
    return out5.transpose(0, 2, 4, 1, 3).reshape(m, m, out_dim)


# speedup vs baseline: 1.0298x; 1.0298x over previous
import functools

import numpy as np

import jax
import jax.numpy as jnp
from jax import lax
from jax.experimental import pallas as pl
from jax.experimental.pallas import tpu as pltpu
from jax.experimental.pallas import tpu_sc as plsc

_NC = 2
_NS = 16
_NW = _NC * _NS

_BI = 128
_BJ = 512
_PK = 4
_CI = 32


def _sc_gather_g2(table, idx4):
    m = idx4.shape[1]
    d = table.shape[1]
    bpw = m // _NS
    mesh = plsc.VectorSubcoreMesh(core_axis_name="c", subcore_axis_name="s")

    @functools.partial(
        pl.kernel,
        mesh=mesh,
        out_type=[
            jax.ShapeDtypeStruct((m, 2 * d), table.dtype),
            jax.ShapeDtypeStruct((m, 2 * d), table.dtype),
        ],
        scratch_types=[
            pltpu.VMEM((bpw,), jnp.int32),
            pltpu.VMEM((bpw, d), table.dtype),
            pltpu.SemaphoreType.DMA,
        ],
    )
    def gather_kernel(table_hbm, idx4_hbm, g_hbm, gp_hbm, idx_v, rows_v, sem):
        half = lax.axis_index("c")
        rbase = lax.axis_index("s") * bpw

        @pl.when(half == 0)
        def _():
            for src, out_hbm in ((0, g_hbm), (2, gp_hbm)):
                pltpu.sync_copy(idx4_hbm.at[src, pl.ds(rbase, bpw)], idx_v)
                pltpu.async_copy(table_hbm.at[idx_v], rows_v, sem).wait()
                pltpu.sync_copy(rows_v,
                                out_hbm.at[pl.ds(rbase, bpw), pl.ds(0, d)])

        @pl.when(half == 1)
        def _():
            for src, out_hbm in ((1, g_hbm), (3, gp_hbm)):
                pltpu.sync_copy(idx4_hbm.at[src, pl.ds(rbase, bpw)], idx_v)
                pltpu.async_copy(table_hbm.at[idx_v], rows_v, sem).wait()
                pltpu.sync_copy(rows_v,
                                out_hbm.at[pl.ds(rbase, bpw), pl.ds(d, d)])

    return gather_kernel(table, idx4)


def _pair_ffnn_body(gi_ref, gjp_ref, wlw_ref, wrbd_ref, b1w_ref, woutp_ref,
                    boutp_ref, out_ref):
    hw = wlw_ref.shape[1]
    ow = woutp_ref.shape[1]
    bjp = gjp_ref.shape[0]
    l = jnp.dot(gi_ref[...], wlw_ref[...], preferred_element_type=jnp.float32)
    l = l + b1w_ref[...]
    r = jnp.dot(gjp_ref[...], wrbd_ref[...], preferred_element_type=jnp.float32)
    for c in range(_BI // _CI):
        lc = l[c * _CI:(c + 1) * _CI]
        h = jnp.maximum(lc[:, None, :] + r[None, :, :], 0.0)
        o = jnp.dot(h.reshape(_CI * bjp, hw), woutp_ref[...],
                    preferred_element_type=jnp.float32)
        o = o + boutp_ref[...]
        out_ref[pl.ds(c * _CI, _CI)] = jnp.transpose(
            o.reshape(_CI, bjp, ow), (0, 2, 1))


def kernel(subword_embeddings, begin_indexes, end_indexes, W1, b1, Wout, bout):
    m = begin_indexes.shape[0]
    d = subword_embeddings.shape[1]
    h_dim = W1.shape[1]
    out_dim = Wout.shape[1]
    pk = _PK
    grp = m // pk

    idx_n = jnp.stack([begin_indexes, end_indexes]).astype(jnp.int32)
    idx_p = idx_n.reshape(2, pk, grp).transpose(0, 2, 1).reshape(2, m)
    idx4 = jnp.concatenate([idx_n, idx_p])
    g, gp = _sc_gather_g2(subword_embeddings, idx4)
    gjp = gp.reshape(grp, pk * 2 * d)

    wl = W1[: 2 * d]
    wr = W1[2 * d:]
    wl_wide = jnp.concatenate([wl] * pk, axis=1)
    wr_bd = jax.scipy.linalg.block_diag(*([wr] * pk))
    wout_bd = jax.scipy.linalg.block_diag(*([Wout] * pk))
    b1_wide = jnp.concatenate([b1] * pk).reshape(1, pk * h_dim)
    bout_wide = jnp.concatenate([bout] * pk).reshape(1, pk * out_dim)
    cols = np.arange(pk * out_dim)
    kk = 8 * (cols // (8 * pk)) + cols % 8
    tt = (cols % (8 * pk)) // 8
    perm = tt * out_dim + kk
    wout_p = wout_bd[:, perm]
    bout_p = bout_wide[:, perm]

    grid = (m // _BI, m // _BJ)
    out = pl.pallas_call(
        _pair_ffnn_body,
        grid=grid,
        in_specs=[
            pl.BlockSpec((_BI, 2 * d), lambda i, j: (i, 0)),
            pl.BlockSpec((_BJ // pk, pk * 2 * d), lambda i, j: (j, 0)),
            pl.BlockSpec((2 * d, pk * h_dim), lambda i, j: (0, 0)),
            pl.BlockSpec((pk * 2 * d, pk * h_dim), lambda i, j: (0, 0)),
            pl.BlockSpec((1, pk * h_dim), lambda i, j: (0, 0)),
            pl.BlockSpec((pk * h_dim, pk * out_dim), lambda i, j: (0, 0)),
            pl.BlockSpec((1, pk * out_dim), lambda i, j: (0, 0)),
        ],
        out_specs=pl.BlockSpec((_BI, pk * out_dim, _BJ // pk),
                               lambda i, j: (i, 0, j)),
        out_shape=jax.ShapeDtypeStruct((m, pk * out_dim, grp), jnp.float32),
    )(g, gjp, wl_wide, wr_bd, b1_wide, wout_p, bout_p)
    out5 = out.reshape(m, out_dim // 8, pk, 8, grp)
    return out5.transpose(0, 2, 4, 1, 3).reshape(m, m, out_dim)


# speedup vs baseline: 1.1682x; 1.1344x over previous
import functools

import jax
import jax.numpy as jnp
from jax import lax
from jax.experimental import pallas as pl
from jax.experimental.pallas import tpu as pltpu
from jax.experimental.pallas import tpu_sc as plsc

_NC = 2
_NS = 16
_NW = _NC * _NS

_BI = 128
_PK = 4
_CI = 32


def _sc_gather_both(table, idx2):
    m = idx2.shape[1]
    d = table.shape[1]
    bpw = m // _NS
    gbw = bpw // _PK
    grp = m // _PK
    mesh = plsc.VectorSubcoreMesh(core_axis_name="c", subcore_axis_name="s")

    @functools.partial(
        pl.kernel,
        mesh=mesh,
        out_type=[
            jax.ShapeDtypeStruct((m, 2 * d), table.dtype),
            jax.ShapeDtypeStruct((grp, 2 * _PK * d), table.dtype),
        ],
        scratch_types=[
            pltpu.VMEM((bpw,), jnp.int32),
            pltpu.VMEM((bpw, d), table.dtype),
            pltpu.VMEM((bpw,), jnp.int32),
            pltpu.VMEM((bpw, d), table.dtype),
            pltpu.SemaphoreType.DMA,
        ],
    )
    def gather_kernel(table_hbm, idx2_hbm, g_hbm, gjp_hbm,
                      idx_v, rows_v, idxp_v, rowsp_v, sem):
        half = lax.axis_index("c")
        s = lax.axis_index("s")
        rbase = s * bpw
        gbase = s * gbw

        def do(row, coff):
            pltpu.sync_copy(idx2_hbm.at[row, pl.ds(rbase, bpw)], idx_v)
            pltpu.async_copy(table_hbm.at[idx_v], rows_v, sem).wait()
            pltpu.sync_copy(rows_v,
                            g_hbm.at[pl.ds(rbase, bpw), pl.ds(coff, d)])
            for t in range(_PK):
                pltpu.sync_copy(idx2_hbm.at[row, pl.ds(t * grp + gbase, gbw)],
                                idxp_v.at[pl.ds(t * gbw, gbw)])
            pltpu.async_copy(table_hbm.at[idxp_v], rowsp_v, sem).wait()
            for t in range(_PK):
                pltpu.sync_copy(
                    rowsp_v.at[pl.ds(t * gbw, gbw)],
                    gjp_hbm.at[pl.ds(gbase, gbw),
                               pl.ds(t * 2 * d + coff, d)])

        @pl.when(half == 0)
        def _():
            do(0, 0)

        @pl.when(half == 1)
        def _():
            do(1, d)

    return gather_kernel(table, idx2)


def _pair_ffnn_body(gi_ref, gjp_ref, w1_ref, b1_ref, wout_ref, bout_ref,
                    out_ref, wlw_s, wrbd_s, b1w_s, woutp_s, boutp_s):
    d2 = w1_ref.shape[0] // 2
    h = w1_ref.shape[1]
    od = wout_ref.shape[1]
    bjp = gjp_ref.shape[0]
    hw = _PK * h
    ow = _PK * od
    ncb = ow // 8

    @pl.when(pl.program_id(0) == 0)
    def _prep():
        w1 = w1_ref[...]
        wl = w1[:d2]
        wr = w1[d2:]
        wlw_s[...] = jnp.concatenate([wl] * _PK, axis=1)
        zr = jnp.zeros_like(wr)
        for t in range(_PK):
            wrbd_s[pl.ds(t * d2, d2)] = jnp.concatenate(
                [wr if u == t else zr for u in range(_PK)], axis=1)
        b1w_s[...] = jnp.concatenate([b1_ref[...]] * _PK, axis=1)
        wout = wout_ref[...]
        zo = jnp.zeros((h, 8), jnp.float32)
        for t in range(_PK):
            woutp_s[pl.ds(t * h, h)] = jnp.concatenate(
                [wout[:, (cb // _PK) * 8:(cb // _PK) * 8 + 8]
                 if cb % _PK == t else zo for cb in range(ncb)], axis=1)
        bo = bout_ref[...]
        boutp_s[...] = jnp.concatenate(
            [bo[:, (cb // _PK) * 8:(cb // _PK) * 8 + 8] for cb in range(ncb)],
            axis=1)

    l = jnp.dot(gi_ref[...], wlw_s[...], preferred_element_type=jnp.float32)
    l = l + b1w_s[...]
    r = jnp.dot(gjp_ref[...], wrbd_s[...], preferred_element_type=jnp.float32)
    for c in range(_BI // _CI):
        lc = l[c * _CI:(c + 1) * _CI]
        hh = jnp.maximum(lc[:, None, :] + r[None, :, :], 0.0)
        o = jnp.dot(hh.reshape(_CI * bjp, hw), woutp_s[...],
                    preferred_element_type=jnp.float32)
        o = o + boutp_s[...]
        out_ref[pl.ds(c * _CI, _CI)] = jnp.transpose(
            o.reshape(_CI, bjp, ow), (0, 2, 1))


def kernel(subword_embeddings, begin_indexes, end_indexes, W1, b1, Wout, bout):
    m = begin_indexes.shape[0]
    d = subword_embeddings.shape[1]
    h_dim = W1.shape[1]
    out_dim = Wout.shape[1]
    pk = _PK
    grp = m // pk

    idx2 = jnp.stack([begin_indexes, end_indexes]).astype(jnp.int32)
    g, gjp = _sc_gather_both(subword_embeddings, idx2)

    grid = (m // _BI, 1)
    out = pl.pallas_call(
        _pair_ffnn_body,
        grid=grid,
        in_specs=[
            pl.BlockSpec((_BI, 2 * d), lambda i, j: (i, 0)),
            pl.BlockSpec((grp, pk * 2 * d), lambda i, j: (0, 0)),
            pl.BlockSpec((4 * d, h_dim), lambda i, j: (0, 0)),
            pl.BlockSpec((1, h_dim), lambda i, j: (0, 0)),
            pl.BlockSpec((h_dim, out_dim), lambda i, j: (0, 0)),
            pl.BlockSpec((1, out_dim), lambda i, j: (0, 0)),
        ],
        out_specs=pl.BlockSpec((_BI, pk * out_dim, grp), lambda i, j: (i, 0, j)),
        out_shape=jax.ShapeDtypeStruct((m, pk * out_dim, grp), jnp.float32),
        scratch_shapes=[
            pltpu.VMEM((2 * d, pk * h_dim), jnp.float32),
            pltpu.VMEM((pk * 2 * d, pk * h_dim), jnp.float32),
            pltpu.VMEM((1, pk * h_dim), jnp.float32),
            pltpu.VMEM((pk * h_dim, pk * out_dim), jnp.float32),
            pltpu.VMEM((1, pk * out_dim), jnp.float32),
        ],
    )(g, gjp, W1, b1.reshape(1, h_dim), Wout, bout.reshape(1, out_dim))
    out5 = out.reshape(m, out_dim // 8, pk, 8, grp)
    return out5.transpose(0, 2, 4, 1, 3).reshape(m, m, out_dim)
